# Initial kernel scaffold; baseline (speedup 1.0000x reference)
import jax
import jax.numpy as jnp
from jax.experimental import pallas as pl


def kernel(index, image):
    raise NotImplementedError("write your pallas kernel here")


# speedup vs baseline: 1.8948x; 1.8948x over previous
import functools

import jax
import jax.numpy as jnp
from jax import lax
from jax.experimental import pallas as pl
from jax.experimental.pallas import tpu as pltpu
from jax.experimental.pallas import tpu_sc as plsc

_NUM_CORES = 2
_NUM_SUBCORES = 16
_NW = _NUM_CORES * _NUM_SUBCORES

_CH = 8


def _gather_rows(flat_idx, table, *, B, D):
    b_per_w = B // _NW
    nch = b_per_w // _CH

    mesh = plsc.VectorSubcoreMesh(core_axis_name="c", subcore_axis_name="s")

    @functools.partial(
        pl.kernel,
        mesh=mesh,
        out_type=jax.ShapeDtypeStruct((B, D), jnp.float32),
        scratch_types=[
            pltpu.VMEM((b_per_w,), jnp.int32),
            pltpu.VMEM((2, _CH, D), jnp.float32),
            pltpu.SemaphoreType.DMA,
            pltpu.SemaphoreType.DMA,
            pltpu.SemaphoreType.DMA,
            pltpu.SemaphoreType.DMA,
        ],
    )
    def body(idx_hbm, table_hbm, out_hbm, idx_v, buf, g0, g1, s0, s1):
        wid = lax.axis_index("s") * _NUM_CORES + lax.axis_index("c")
        base = wid * b_per_w
        pltpu.sync_copy(idx_hbm.at[pl.ds(base, b_per_w)], idx_v)

        gsem = (g0, g1)
        ssem = (s0, s1)

        def issue_gather(i):
            p = i % 2
            return pltpu.async_copy(
                table_hbm.at[idx_v.at[pl.ds(i * _CH, _CH)]],
                buf.at[p],
                gsem[p],
            )

        def issue_write(i):
            p = i % 2
            return pltpu.async_copy(
                buf.at[p],
                out_hbm.at[pl.ds(base + i * _CH, _CH)],
                ssem[p],
            )

        g_next = issue_gather(0)
        w_prev = None
        for i in range(nch):
            g_cur = g_next
            if i + 1 < nch:
                if w_prev is not None:
                    w_prev.wait()
                g_next = issue_gather(i + 1)
            g_cur.wait()
            w_prev = issue_write(i)
        w_prev.wait()

    return body(flat_idx, table)


def kernel(index, image):
    b, mn, ws2, c = image.shape
    _, Nq, topw = index.shape
    D = ws2 * c
    B = b * Nq * topw

    table = image.reshape(b * mn, D)
    offs = (jnp.arange(b, dtype=jnp.int32) * mn)[:, None, None]
    flat_idx = (index.astype(jnp.int32) + offs).reshape(B)

    out = _gather_rows(flat_idx, table, B=B, D=D)
    return out.reshape(b, Nq, topw, ws2, c)
